# baseline (speedup 1.0000x reference)
import functools

import jax
import jax.numpy as jnp
from jax import lax
from jax.experimental import pallas as pl
from jax.experimental.pallas import tpu as pltpu
from jax.experimental.pallas import tpu_sc as plsc

N_NODES = 10000
N_EDGES = 320000
D = 128

NC = 2
NS = 16
NW = NC * NS

B = 128
K = 80
KW = 8
EPT = B * K
E_PAD = NW * EPT
ACC_N = 10240
SLAB = ACC_N // NS


def _seg_body(with_counts, x_hbm, src_hbm, dst_hbm, out_hbm, cnt_hbm,
              rows0, rows1, src_v, dst_v, ones_v, zc_v, acc_sh, cnt_sh,
              s0, s1):
    _ZERO16 = jnp.zeros((16,), jnp.float32)
    _ONES16 = jnp.ones((16,), jnp.float32)
    cid = lax.axis_index("c")
    sid = lax.axis_index("s")
    wid = sid * NC + cid
    base = sid * SLAB

    def zr(i, _):
        rows0[i // 8, pl.ds((i % 8) * 16, 16)] = _ZERO16
        return 0
    lax.fori_loop(0, B * 8, zr, 0)
    for k in range(SLAB // B):
        pltpu.sync_copy(rows0, acc_sh.at[pl.ds(base + k * B, B)])

    if with_counts:
        def zo(i, _):
            ones_v[pl.ds(i * 16, 16)] = _ONES16
            return 0
        lax.fori_loop(0, B // 16, zo, 0)

        def zs(i, _):
            zc_v[pl.ds(i * 16, 16)] = _ZERO16
            return 0
        lax.fori_loop(0, SLAB // 16, zs, 0)
        pltpu.sync_copy(zc_v, cnt_sh.at[pl.ds(sid * SLAB, SLAB)])

    plsc.subcore_barrier()

    for h in range(K // KW):
        pltpu.sync_copy(src_hbm.at[wid, pl.ds(h * KW, KW)], src_v)
        pltpu.sync_copy(dst_hbm.at[wid, pl.ds(h * KW, KW)], dst_v)
        pltpu.async_copy(x_hbm.at[src_v.at[0]], rows0, s0)
        pltpu.async_copy(x_hbm.at[src_v.at[1]], rows1, s1)

        def body(i, _):
            for par, rows, sem in ((0, rows0, s0), (1, rows1, s1)):
                j = 2 * i + par
                pltpu.make_async_copy(x_hbm.at[src_v.at[j]], rows, sem).wait()
                if with_counts:
                    pltpu.sync_copy(ones_v, cnt_sh.at[dst_v.at[j]], add=True)
                pltpu.sync_copy(rows, acc_sh.at[dst_v.at[j]], add=True)

                @pl.when(j + 2 < KW)
                def _():
                    pltpu.async_copy(x_hbm.at[src_v.at[j + 2]], rows, sem)
            return 0
        lax.fori_loop(0, KW // 2, body, 0)

    plsc.subcore_barrier()

    pltpu.sync_copy(acc_sh.at[pl.ds(base, SLAB)],
                    out_hbm.at[cid, pl.ds(base, SLAB)])
    if with_counts:
        pltpu.sync_copy(cnt_sh.at[pl.ds(sid * SLAB, SLAB)],
                        cnt_hbm.at[cid, pl.ds(sid * SLAB, SLAB)])


def _make_seg(with_counts):
    return functools.partial(
        pl.kernel,
        out_type=(
            jax.ShapeDtypeStruct((NC, ACC_N, D), jnp.float32),
            jax.ShapeDtypeStruct((NC, ACC_N), jnp.float32),
        ),
        mesh=plsc.VectorSubcoreMesh(core_axis_name="c", subcore_axis_name="s"),
        scratch_types=[
            pltpu.VMEM((B, D), jnp.float32),
            pltpu.VMEM((B, D), jnp.float32),
            pltpu.VMEM((KW, B), jnp.int32),
            pltpu.VMEM((KW, B), jnp.int32),
            pltpu.VMEM((B,), jnp.float32),
            pltpu.VMEM((SLAB,), jnp.float32),
            pltpu.VMEM_SHARED((ACC_N, D), jnp.float32),
            pltpu.VMEM_SHARED((ACC_N,), jnp.float32),
            pltpu.SemaphoreType.DMA,
            pltpu.SemaphoreType.DMA,
        ],
    )(functools.partial(_seg_body, with_counts))


_seg_sum_cnt = _make_seg(True)
_seg_sum = _make_seg(False)


RB = 512


def _lin_body(x_ref, w_ref, b_ref, o_ref):
    o_ref[...] = (jnp.dot(x_ref[...], w_ref[...],
                          preferred_element_type=jnp.float32) + b_ref[...])


def _lin(x, w, b):
    return pl.pallas_call(
        _lin_body,
        grid=(ACC_N // RB,),
        in_specs=[
            pl.BlockSpec((RB, D), lambda i: (i, 0)),
            pl.BlockSpec((D, D), lambda i: (0, 0)),
            pl.BlockSpec((1, D), lambda i: (0, 0)),
        ],
        out_specs=pl.BlockSpec((RB, D), lambda i: (i, 0)),
        out_shape=jax.ShapeDtypeStruct((ACC_N, D), jnp.float32),
    )(x, w, b)


def _comb_body(relu, p_ref, cnt_ref, xr_ref, wl_ref, o_ref):
    s = p_ref[0] + p_ref[1]
    c = cnt_ref[0] + cnt_ref[1]
    agg = s * (1.0 / jnp.maximum(c, 1.0))[:, None]
    y = jnp.dot(agg, wl_ref[...],
                preferred_element_type=jnp.float32) + xr_ref[...]
    o_ref[...] = jnp.maximum(y, 0.0) if relu else y


def _comb(p, cnt, xr, wl, relu):
    return pl.pallas_call(
        functools.partial(_comb_body, relu),
        grid=(ACC_N // RB,),
        in_specs=[
            pl.BlockSpec((NC, RB, D), lambda i: (0, i, 0)),
            pl.BlockSpec((NC, RB), lambda i: (0, i)),
            pl.BlockSpec((RB, D), lambda i: (i, 0)),
            pl.BlockSpec((D, D), lambda i: (0, 0)),
        ],
        out_specs=pl.BlockSpec((RB, D), lambda i: (i, 0)),
        out_shape=jax.ShapeDtypeStruct((ACC_N, D), jnp.float32),
    )(p, cnt, xr, wl)


def kernel(x, edge_index, W1_l, W1_r, b1, W2_l, W2_r, b2):
    src = edge_index[0].astype(jnp.int32)
    dst = edge_index[1].astype(jnp.int32)
    ppt = EPT - N_EDGES // NW
    pad_src = jnp.zeros((NW, ppt), jnp.int32)
    pad_dst = jnp.broadcast_to(
        N_NODES + jnp.arange(ppt, dtype=jnp.int32), (NW, ppt))
    src_p = jnp.concatenate(
        [src.reshape(NW, -1), pad_src], axis=1).reshape(NW, K, B)
    dst_p = jnp.concatenate(
        [dst.reshape(NW, -1), pad_dst], axis=1).reshape(NW, K, B)
    x_pad = jnp.concatenate(
        [x, jnp.zeros((ACC_N - N_NODES, D), x.dtype)], axis=0)
    b1r = b1.reshape(1, D)
    b2r = b2.reshape(1, D)

    xr1 = _lin(x_pad, W1_r, b1r)
    p1, cnt = _seg_sum_cnt(x_pad, src_p, dst_p)
    h = _comb(p1, cnt, xr1, W1_l, relu=True)
    xr2 = _lin(h, W2_r, b2r)
    p2, _ = _seg_sum(h, src_p, dst_p)
    out = _comb(p2, cnt, xr2, W2_l, relu=False)
    return out[:N_NODES]


# speedup vs baseline: 1.0263x; 1.0263x over previous
import functools

import jax
import jax.numpy as jnp
from jax import lax
from jax.experimental import pallas as pl
from jax.experimental.pallas import tpu as pltpu
from jax.experimental.pallas import tpu_sc as plsc

N_NODES = 10000
N_EDGES = 320000
D = 128

NC = 2
NS = 16
NW = NC * NS

B = 80
K = 128
KW = 64
EPT = B * K
E_PAD = NW * EPT
ACC_N = 10240
SLAB = ACC_N // NS


def _seg_body(with_counts, x_hbm, src_hbm, dst_hbm, out_hbm, cnt_hbm,
              rows0, rows1, src_v, dst_v, ones_v, zc_v, acc_sh, cnt_sh,
              s0, s1):
    _ZERO16 = jnp.zeros((16,), jnp.float32)
    _ONES16 = jnp.ones((16,), jnp.float32)
    cid = lax.axis_index("c")
    sid = lax.axis_index("s")
    wid = sid * NC + cid
    base = sid * SLAB

    def zr(i, _):
        rows0[i // 8, pl.ds((i % 8) * 16, 16)] = _ZERO16
        return 0
    lax.fori_loop(0, B * 8, zr, 0)
    for k in range(SLAB // B):
        pltpu.sync_copy(rows0, acc_sh.at[pl.ds(base + k * B, B)])

    if with_counts:
        def zo(i, _):
            ones_v[pl.ds(i * 16, 16)] = _ONES16
            return 0
        lax.fori_loop(0, B // 16, zo, 0)

        def zs(i, _):
            zc_v[pl.ds(i * 16, 16)] = _ZERO16
            return 0
        lax.fori_loop(0, SLAB // 16, zs, 0)
        pltpu.sync_copy(zc_v, cnt_sh.at[pl.ds(sid * SLAB, SLAB)])

    plsc.subcore_barrier()

    for h in range(K // KW):
        pltpu.sync_copy(src_hbm.at[wid, pl.ds(h * KW, KW)], src_v)
        pltpu.sync_copy(dst_hbm.at[wid, pl.ds(h * KW, KW)], dst_v)
        pltpu.async_copy(x_hbm.at[src_v.at[0]], rows0, s0)
        pltpu.async_copy(x_hbm.at[src_v.at[1]], rows1, s1)

        def body(i, _):
            for par, rows, sem in ((0, rows0, s0), (1, rows1, s1)):
                j = 2 * i + par
                pltpu.make_async_copy(x_hbm.at[src_v.at[j]], rows, sem).wait()
                if with_counts:
                    pltpu.sync_copy(ones_v, cnt_sh.at[dst_v.at[j]], add=True)
                pltpu.sync_copy(rows, acc_sh.at[dst_v.at[j]], add=True)

                @pl.when(j + 2 < KW)
                def _():
                    pltpu.async_copy(x_hbm.at[src_v.at[j + 2]], rows, sem)
            return 0
        lax.fori_loop(0, KW // 2, body, 0)

    plsc.subcore_barrier()

    pltpu.sync_copy(acc_sh.at[pl.ds(base, SLAB)],
                    out_hbm.at[cid, pl.ds(base, SLAB)])
    if with_counts:
        pltpu.sync_copy(cnt_sh.at[pl.ds(sid * SLAB, SLAB)],
                        cnt_hbm.at[cid, pl.ds(sid * SLAB, SLAB)])


def _make_seg(with_counts):
    return functools.partial(
        pl.kernel,
        out_type=(
            jax.ShapeDtypeStruct((NC, ACC_N, D), jnp.float32),
            jax.ShapeDtypeStruct((NC, ACC_N), jnp.float32),
        ),
        mesh=plsc.VectorSubcoreMesh(core_axis_name="c", subcore_axis_name="s"),
        scratch_types=[
            pltpu.VMEM((B, D), jnp.float32),
            pltpu.VMEM((B, D), jnp.float32),
            pltpu.VMEM((KW, B), jnp.int32),
            pltpu.VMEM((KW, B), jnp.int32),
            pltpu.VMEM((B,), jnp.float32),
            pltpu.VMEM((SLAB,), jnp.float32),
            pltpu.VMEM_SHARED((ACC_N, D), jnp.float32),
            pltpu.VMEM_SHARED((ACC_N,), jnp.float32),
            pltpu.SemaphoreType.DMA,
            pltpu.SemaphoreType.DMA,
        ],
    )(functools.partial(_seg_body, with_counts))


_seg_sum_cnt = _make_seg(True)
_seg_sum = _make_seg(False)


RB = 512
_GRID = (N_NODES + RB - 1) // RB


def _lin_body(x_ref, w_ref, b_ref, o_ref):
    o_ref[...] = (jnp.dot(x_ref[...], w_ref[...],
                          preferred_element_type=jnp.float32) + b_ref[...])


def _lin(x, w, b):
    return pl.pallas_call(
        _lin_body,
        grid=(_GRID,),
        in_specs=[
            pl.BlockSpec((RB, D), lambda i: (i, 0)),
            pl.BlockSpec((D, D), lambda i: (0, 0)),
            pl.BlockSpec((1, D), lambda i: (0, 0)),
        ],
        out_specs=pl.BlockSpec((RB, D), lambda i: (i, 0)),
        out_shape=jax.ShapeDtypeStruct((N_NODES, D), jnp.float32),
    )(x, w, b)


def _agg_of(p_ref, cnt_ref):
    s = p_ref[0] + p_ref[1]
    c = cnt_ref[0] + cnt_ref[1]
    return s * (1.0 / jnp.maximum(c, 1.0))[:, None]


def _comb1_body(p_ref, cnt_ref, xr_ref, wl_ref, w2_ref, b2_ref,
                h_ref, xr2_ref):
    agg = _agg_of(p_ref, cnt_ref)
    h = jnp.maximum(
        jnp.dot(agg, wl_ref[...], preferred_element_type=jnp.float32)
        + xr_ref[...], 0.0)
    h_ref[...] = h
    xr2_ref[...] = (jnp.dot(h, w2_ref[...],
                            preferred_element_type=jnp.float32) + b2_ref[...])


def _comb1(p, cnt, xr, wl, w2, b2):
    return pl.pallas_call(
        _comb1_body,
        grid=(_GRID,),
        in_specs=[
            pl.BlockSpec((NC, RB, D), lambda i: (0, i, 0)),
            pl.BlockSpec((NC, RB), lambda i: (0, i)),
            pl.BlockSpec((RB, D), lambda i: (i, 0)),
            pl.BlockSpec((D, D), lambda i: (0, 0)),
            pl.BlockSpec((D, D), lambda i: (0, 0)),
            pl.BlockSpec((1, D), lambda i: (0, 0)),
        ],
        out_specs=[
            pl.BlockSpec((RB, D), lambda i: (i, 0)),
            pl.BlockSpec((RB, D), lambda i: (i, 0)),
        ],
        out_shape=[
            jax.ShapeDtypeStruct((N_NODES, D), jnp.float32),
            jax.ShapeDtypeStruct((N_NODES, D), jnp.float32),
        ],
    )(p, cnt, xr, wl, w2, b2)


def _comb2_body(p_ref, cnt_ref, xr_ref, wl_ref, o_ref):
    agg = _agg_of(p_ref, cnt_ref)
    o_ref[...] = jnp.dot(agg, wl_ref[...],
                         preferred_element_type=jnp.float32) + xr_ref[...]


def _comb2(p, cnt, xr, wl):
    return pl.pallas_call(
        _comb2_body,
        grid=(_GRID,),
        in_specs=[
            pl.BlockSpec((NC, RB, D), lambda i: (0, i, 0)),
            pl.BlockSpec((NC, RB), lambda i: (0, i)),
            pl.BlockSpec((RB, D), lambda i: (i, 0)),
            pl.BlockSpec((D, D), lambda i: (0, 0)),
        ],
        out_specs=pl.BlockSpec((RB, D), lambda i: (i, 0)),
        out_shape=jax.ShapeDtypeStruct((N_NODES, D), jnp.float32),
    )(p, cnt, xr, wl)


def kernel(x, edge_index, W1_l, W1_r, b1, W2_l, W2_r, b2):
    src = edge_index[0].astype(jnp.int32)
    dst = edge_index[1].astype(jnp.int32)
    ppt = EPT - N_EDGES // NW
    pad_src = jnp.zeros((NW, ppt), jnp.int32)
    pad_dst = jnp.broadcast_to(
        N_NODES + jnp.arange(ppt, dtype=jnp.int32), (NW, ppt))
    src_p = jnp.concatenate(
        [src.reshape(NW, -1), pad_src], axis=1).reshape(NW, K, B)
    dst_p = jnp.concatenate(
        [dst.reshape(NW, -1), pad_dst], axis=1).reshape(NW, K, B)
    b1r = b1.reshape(1, D)
    b2r = b2.reshape(1, D)

    xr1 = _lin(x, W1_r, b1r)
    p1, cnt = _seg_sum_cnt(x, src_p, dst_p)
    h, xr2 = _comb1(p1, cnt, xr1, W1_l, W2_r, b2r)
    p2, _ = _seg_sum(h, src_p, dst_p)
    return _comb2(p2, cnt, xr2, W2_l)


# speedup vs baseline: 3.0129x; 2.9356x over previous
import functools

import jax
import jax.numpy as jnp
from jax import lax
from jax.experimental import pallas as pl
from jax.experimental.pallas import tpu as pltpu
from jax.experimental.pallas import tpu_sc as plsc

N_NODES = 10000
N_EDGES = 320000
D = 128

NC = 2
NS = 16
NW = NC * NS

B = 100
K = 100
KW = 50
EPT = B * K
E_PAD = NW * EPT
ACC_N = 10240
SLAB = ACC_N // NS


def _seg_body(with_counts, x_hbm, src_hbm, dst_hbm, out_hbm, cnt_hbm,
              rows0, rows1, src_v, dst_v, ones_v, zc_v, acc_sh, cnt_sh,
              s0, s1):
    _ZERO16 = jnp.zeros((16,), jnp.float32)
    _ONES16 = jnp.ones((16,), jnp.float32)
    cid = lax.axis_index("c")
    sid = lax.axis_index("s")
    wid = sid * NC + cid
    base = sid * SLAB

    def zr(i, _):
        rows0[i // 8, pl.ds((i % 8) * 16, 16)] = _ZERO16
        return 0
    lax.fori_loop(0, B * 8, zr, 0)
    nfull, rem = divmod(SLAB, B)
    for k in range(nfull):
        pltpu.sync_copy(rows0, acc_sh.at[pl.ds(base + k * B, B)])
    if rem:
        pltpu.sync_copy(rows0.at[pl.ds(0, rem)],
                        acc_sh.at[pl.ds(base + nfull * B, rem)])

    if with_counts:
        def zo(i, _):
            ones_v[pl.ds(jnp.minimum(i * 16, B - 16), 16)] = _ONES16
            return 0
        lax.fori_loop(0, (B + 15) // 16, zo, 0)

        def zs(i, _):
            zc_v[pl.ds(i * 16, 16)] = _ZERO16
            return 0
        lax.fori_loop(0, SLAB // 16, zs, 0)
        pltpu.sync_copy(zc_v, cnt_sh.at[pl.ds(sid * SLAB, SLAB)])

    plsc.subcore_barrier()

    for h in range(K // KW):
        pltpu.sync_copy(src_hbm.at[wid, h], src_v)
        pltpu.sync_copy(dst_hbm.at[wid, h], dst_v)
        pltpu.async_copy(x_hbm.at[src_v.at[0]], rows0, s0)
        pltpu.async_copy(x_hbm.at[src_v.at[1]], rows1, s1)

        def body(i, _):
            for par, rows, sem in ((0, rows0, s0), (1, rows1, s1)):
                j = 2 * i + par
                pltpu.make_async_copy(x_hbm.at[src_v.at[j]], rows, sem).wait()
                if with_counts:
                    pltpu.sync_copy(ones_v, cnt_sh.at[dst_v.at[j]], add=True)
                pltpu.sync_copy(rows, acc_sh.at[dst_v.at[j]], add=True)

                @pl.when(j + 2 < KW)
                def _():
                    pltpu.async_copy(x_hbm.at[src_v.at[j + 2]], rows, sem)
            return 0
        lax.fori_loop(0, KW // 2, body, 0)

    plsc.subcore_barrier()

    pltpu.sync_copy(acc_sh.at[pl.ds(base, SLAB)],
                    out_hbm.at[cid, pl.ds(base, SLAB)])
    if with_counts:
        pltpu.sync_copy(cnt_sh.at[pl.ds(sid * SLAB, SLAB)],
                        cnt_hbm.at[cid, pl.ds(sid * SLAB, SLAB)])


def _make_seg(with_counts):
    return functools.partial(
        pl.kernel,
        out_type=(
            jax.ShapeDtypeStruct((NC, ACC_N, D), jnp.float32),
            jax.ShapeDtypeStruct((NC, ACC_N), jnp.float32),
        ),
        mesh=plsc.VectorSubcoreMesh(core_axis_name="c", subcore_axis_name="s"),
        scratch_types=[
            pltpu.VMEM((B, D), jnp.float32),
            pltpu.VMEM((B, D), jnp.float32),
            pltpu.VMEM((KW, B), jnp.int32),
            pltpu.VMEM((KW, B), jnp.int32),
            pltpu.VMEM((B,), jnp.float32),
            pltpu.VMEM((SLAB,), jnp.float32),
            pltpu.VMEM_SHARED((ACC_N, D), jnp.float32),
            pltpu.VMEM_SHARED((ACC_N,), jnp.float32),
            pltpu.SemaphoreType.DMA,
            pltpu.SemaphoreType.DMA,
        ],
    )(functools.partial(_seg_body, with_counts))


_seg_sum_cnt = _make_seg(True)
_seg_sum = _make_seg(False)


RB = 512
_GRID = (N_NODES + RB - 1) // RB


def _lin_body(x_ref, w_ref, b_ref, o_ref):
    o_ref[...] = (jnp.dot(x_ref[...], w_ref[...],
                          preferred_element_type=jnp.float32) + b_ref[...])


def _lin(x, w, b):
    return pl.pallas_call(
        _lin_body,
        grid=(_GRID,),
        in_specs=[
            pl.BlockSpec((RB, D), lambda i: (i, 0)),
            pl.BlockSpec((D, D), lambda i: (0, 0)),
            pl.BlockSpec((1, D), lambda i: (0, 0)),
        ],
        out_specs=pl.BlockSpec((RB, D), lambda i: (i, 0)),
        out_shape=jax.ShapeDtypeStruct((N_NODES, D), jnp.float32),
    )(x, w, b)


def _agg_of(p_ref, cnt_ref):
    s = p_ref[0] + p_ref[1]
    c = cnt_ref[0] + cnt_ref[1]
    return s * (1.0 / jnp.maximum(c, 1.0))[:, None]


def _comb1_body(p_ref, cnt_ref, xr_ref, wl_ref, w2_ref, b2_ref,
                h_ref, xr2_ref):
    agg = _agg_of(p_ref, cnt_ref)
    h = jnp.maximum(
        jnp.dot(agg, wl_ref[...], preferred_element_type=jnp.float32)
        + xr_ref[...], 0.0)
    h_ref[...] = h
    xr2_ref[...] = (jnp.dot(h, w2_ref[...],
                            preferred_element_type=jnp.float32) + b2_ref[...])


def _comb1(p, cnt, xr, wl, w2, b2):
    return pl.pallas_call(
        _comb1_body,
        grid=(_GRID,),
        in_specs=[
            pl.BlockSpec((NC, RB, D), lambda i: (0, i, 0)),
            pl.BlockSpec((NC, RB), lambda i: (0, i)),
            pl.BlockSpec((RB, D), lambda i: (i, 0)),
            pl.BlockSpec((D, D), lambda i: (0, 0)),
            pl.BlockSpec((D, D), lambda i: (0, 0)),
            pl.BlockSpec((1, D), lambda i: (0, 0)),
        ],
        out_specs=[
            pl.BlockSpec((RB, D), lambda i: (i, 0)),
            pl.BlockSpec((RB, D), lambda i: (i, 0)),
        ],
        out_shape=[
            jax.ShapeDtypeStruct((N_NODES, D), jnp.float32),
            jax.ShapeDtypeStruct((N_NODES, D), jnp.float32),
        ],
    )(p, cnt, xr, wl, w2, b2)


def _comb2_body(p_ref, cnt_ref, xr_ref, wl_ref, o_ref):
    agg = _agg_of(p_ref, cnt_ref)
    o_ref[...] = jnp.dot(agg, wl_ref[...],
                         preferred_element_type=jnp.float32) + xr_ref[...]


def _comb2(p, cnt, xr, wl):
    return pl.pallas_call(
        _comb2_body,
        grid=(_GRID,),
        in_specs=[
            pl.BlockSpec((NC, RB, D), lambda i: (0, i, 0)),
            pl.BlockSpec((NC, RB), lambda i: (0, i)),
            pl.BlockSpec((RB, D), lambda i: (i, 0)),
            pl.BlockSpec((D, D), lambda i: (0, 0)),
        ],
        out_specs=pl.BlockSpec((RB, D), lambda i: (i, 0)),
        out_shape=jax.ShapeDtypeStruct((N_NODES, D), jnp.float32),
    )(p, cnt, xr, wl)


def kernel(x, edge_index, W1_l, W1_r, b1, W2_l, W2_r, b2):
    src_p = edge_index[0].astype(jnp.int32).reshape(NW, K // KW, KW, B)
    dst_p = edge_index[1].astype(jnp.int32).reshape(NW, K // KW, KW, B)
    b1r = b1.reshape(1, D)
    b2r = b2.reshape(1, D)

    xr1 = _lin(x, W1_r, b1r)
    p1, cnt = _seg_sum_cnt(x, src_p, dst_p)
    h, xr2 = _comb1(p1, cnt, xr1, W1_l, W2_r, b2r)
    p2, _ = _seg_sum(h, src_p, dst_p)
    return _comb2(p2, cnt, xr2, W2_l)


# speedup vs baseline: 3.0871x; 1.0246x over previous
import functools

import jax
import jax.numpy as jnp
from jax import lax
from jax.experimental import pallas as pl
from jax.experimental.pallas import tpu as pltpu
from jax.experimental.pallas import tpu_sc as plsc

N_NODES = 10000
N_EDGES = 320000
D = 128

NC = 2
NS = 16
NW = NC * NS

B = 125
K = 80
KW = 40
EPT = B * K
E_PAD = NW * EPT
ACC_N = 10240
SLAB = ACC_N // NS


def _seg_body(with_counts, x_hbm, src_hbm, dst_hbm, out_hbm, cnt_hbm,
              rows0, rows1, src_v, dst_v, ones_v, zc_v, acc_sh, cnt_sh,
              s0, s1):
    _ZERO16 = jnp.zeros((16,), jnp.float32)
    _ONES16 = jnp.ones((16,), jnp.float32)
    cid = lax.axis_index("c")
    sid = lax.axis_index("s")
    wid = sid * NC + cid
    base = sid * SLAB

    def zr(i, _):
        rows0[i // 8, pl.ds((i % 8) * 16, 16)] = _ZERO16
        return 0
    lax.fori_loop(0, B * 8, zr, 0)
    nfull, rem = divmod(SLAB, B)
    for k in range(nfull):
        pltpu.sync_copy(rows0, acc_sh.at[pl.ds(base + k * B, B)])
    if rem:
        pltpu.sync_copy(rows0.at[pl.ds(0, rem)],
                        acc_sh.at[pl.ds(base + nfull * B, rem)])

    if with_counts:
        def zo(i, _):
            ones_v[pl.ds(jnp.minimum(i * 16, B - 16), 16)] = _ONES16
            return 0
        lax.fori_loop(0, (B + 15) // 16, zo, 0)

        def zs(i, _):
            zc_v[pl.ds(i * 16, 16)] = _ZERO16
            return 0
        lax.fori_loop(0, SLAB // 16, zs, 0)
        pltpu.sync_copy(zc_v, cnt_sh.at[pl.ds(sid * SLAB, SLAB)])

    plsc.subcore_barrier()

    for h in range(K // KW):
        pltpu.sync_copy(src_hbm.at[wid, h], src_v)
        pltpu.sync_copy(dst_hbm.at[wid, h], dst_v)
        pltpu.async_copy(x_hbm.at[src_v.at[0]], rows0, s0)
        pltpu.async_copy(x_hbm.at[src_v.at[1]], rows1, s1)

        def body(i, _):
            for par, rows, sem in ((0, rows0, s0), (1, rows1, s1)):
                j = 2 * i + par
                pltpu.make_async_copy(x_hbm.at[src_v.at[j]], rows, sem).wait()
                if with_counts:
                    pltpu.sync_copy(ones_v, cnt_sh.at[dst_v.at[j]], add=True)
                pltpu.sync_copy(rows, acc_sh.at[dst_v.at[j]], add=True)

                @pl.when(j + 2 < KW)
                def _():
                    pltpu.async_copy(x_hbm.at[src_v.at[j + 2]], rows, sem)
            return 0
        lax.fori_loop(0, KW // 2, body, 0)

    plsc.subcore_barrier()

    pltpu.sync_copy(acc_sh.at[pl.ds(base, SLAB)],
                    out_hbm.at[cid, pl.ds(base, SLAB)])
    if with_counts:
        pltpu.sync_copy(cnt_sh.at[pl.ds(sid * SLAB, SLAB)],
                        cnt_hbm.at[cid, pl.ds(sid * SLAB, SLAB)])


def _make_seg(with_counts):
    return functools.partial(
        pl.kernel,
        out_type=(
            jax.ShapeDtypeStruct((NC, ACC_N, D), jnp.float32),
            jax.ShapeDtypeStruct((NC, ACC_N), jnp.float32),
        ),
        mesh=plsc.VectorSubcoreMesh(core_axis_name="c", subcore_axis_name="s"),
        scratch_types=[
            pltpu.VMEM((B, D), jnp.float32),
            pltpu.VMEM((B, D), jnp.float32),
            pltpu.VMEM((KW, B), jnp.int32),
            pltpu.VMEM((KW, B), jnp.int32),
            pltpu.VMEM((B,), jnp.float32),
            pltpu.VMEM((SLAB,), jnp.float32),
            pltpu.VMEM_SHARED((ACC_N, D), jnp.float32),
            pltpu.VMEM_SHARED((ACC_N,), jnp.float32),
            pltpu.SemaphoreType.DMA,
            pltpu.SemaphoreType.DMA,
        ],
    )(functools.partial(_seg_body, with_counts))


_seg_sum_cnt = _make_seg(True)
_seg_sum = _make_seg(False)


RB = 512
_GRID = (N_NODES + RB - 1) // RB


def _lin_body(x_ref, w_ref, b_ref, o_ref):
    o_ref[...] = (jnp.dot(x_ref[...], w_ref[...],
                          preferred_element_type=jnp.float32) + b_ref[...])


def _lin(x, w, b):
    return pl.pallas_call(
        _lin_body,
        grid=(_GRID,),
        in_specs=[
            pl.BlockSpec((RB, D), lambda i: (i, 0)),
            pl.BlockSpec((D, D), lambda i: (0, 0)),
            pl.BlockSpec((1, D), lambda i: (0, 0)),
        ],
        out_specs=pl.BlockSpec((RB, D), lambda i: (i, 0)),
        out_shape=jax.ShapeDtypeStruct((N_NODES, D), jnp.float32),
    )(x, w, b)


def _agg_of(p_ref, cnt_ref):
    s = p_ref[0] + p_ref[1]
    c = cnt_ref[0] + cnt_ref[1]
    return s * (1.0 / jnp.maximum(c, 1.0))[:, None]


def _comb1_body(p_ref, cnt_ref, xr_ref, wl_ref, w2_ref, b2_ref,
                h_ref, xr2_ref):
    agg = _agg_of(p_ref, cnt_ref)
    h = jnp.maximum(
        jnp.dot(agg, wl_ref[...], preferred_element_type=jnp.float32)
        + xr_ref[...], 0.0)
    h_ref[...] = h
    xr2_ref[...] = (jnp.dot(h, w2_ref[...],
                            preferred_element_type=jnp.float32) + b2_ref[...])


def _comb1(p, cnt, xr, wl, w2, b2):
    return pl.pallas_call(
        _comb1_body,
        grid=(_GRID,),
        in_specs=[
            pl.BlockSpec((NC, RB, D), lambda i: (0, i, 0)),
            pl.BlockSpec((NC, RB), lambda i: (0, i)),
            pl.BlockSpec((RB, D), lambda i: (i, 0)),
            pl.BlockSpec((D, D), lambda i: (0, 0)),
            pl.BlockSpec((D, D), lambda i: (0, 0)),
            pl.BlockSpec((1, D), lambda i: (0, 0)),
        ],
        out_specs=[
            pl.BlockSpec((RB, D), lambda i: (i, 0)),
            pl.BlockSpec((RB, D), lambda i: (i, 0)),
        ],
        out_shape=[
            jax.ShapeDtypeStruct((N_NODES, D), jnp.float32),
            jax.ShapeDtypeStruct((N_NODES, D), jnp.float32),
        ],
    )(p, cnt, xr, wl, w2, b2)


def _comb2_body(p_ref, cnt_ref, xr_ref, wl_ref, o_ref):
    agg = _agg_of(p_ref, cnt_ref)
    o_ref[...] = jnp.dot(agg, wl_ref[...],
                         preferred_element_type=jnp.float32) + xr_ref[...]


def _comb2(p, cnt, xr, wl):
    return pl.pallas_call(
        _comb2_body,
        grid=(_GRID,),
        in_specs=[
            pl.BlockSpec((NC, RB, D), lambda i: (0, i, 0)),
            pl.BlockSpec((NC, RB), lambda i: (0, i)),
            pl.BlockSpec((RB, D), lambda i: (i, 0)),
            pl.BlockSpec((D, D), lambda i: (0, 0)),
        ],
        out_specs=pl.BlockSpec((RB, D), lambda i: (i, 0)),
        out_shape=jax.ShapeDtypeStruct((N_NODES, D), jnp.float32),
    )(p, cnt, xr, wl)


def kernel(x, edge_index, W1_l, W1_r, b1, W2_l, W2_r, b2):
    src_p = edge_index[0].astype(jnp.int32).reshape(NW, K // KW, KW, B)
    dst_p = edge_index[1].astype(jnp.int32).reshape(NW, K // KW, KW, B)
    b1r = b1.reshape(1, D)
    b2r = b2.reshape(1, D)

    xr1 = _lin(x, W1_r, b1r)
    p1, cnt = _seg_sum_cnt(x, src_p, dst_p)
    h, xr2 = _comb1(p1, cnt, xr1, W1_l, W2_r, b2r)
    p2, _ = _seg_sum(h, src_p, dst_p)
    return _comb2(p2, cnt, xr2, W2_l)
